# baseline (speedup 1.0000x reference)
import functools

import jax
import jax.numpy as jnp
from jax import lax
from jax.experimental import pallas as pl
from jax.experimental.pallas import tpu as pltpu
from jax.experimental.pallas import tpu_sc as plsc

N_NODES = 10000
NP = 10240
N_EDGES = 160000
F = 128
R = 4
NC = 2
NS = 16
CHUNK = 80
EDGES_PER_TILE = N_EDGES // NS
CHUNKS_PER_TILE = EDGES_PER_TILE // CHUNK
ROWS_PER_TILE = NP // NS
ZROWS = 80
DEG_CHUNKS = N_EDGES // CHUNK


def _sc_agg_kernel(x_hbm, srcoff_hbm, dst2_hbm, zrow_hbm, ones_hbm,
                   agg_hbm,
                   acc, src_a, src_b, dstbig, rows_a, rows_b,
                   sem_a, sem_b, sem_sa, sem_sb):
    core = lax.axis_index("c")
    sub = lax.axis_index("s")
    edge_base = sub * EDGES_PER_TILE
    row_base = sub * ROWS_PER_TILE

    pltpu.sync_copy(dst2_hbm.at[sub], dstbig)

    def _zero_acc():
        pltpu.sync_copy(zrow_hbm, rows_a)
        for j in range(ROWS_PER_TILE // ZROWS):
            pltpu.sync_copy(rows_a,
                            acc.at[pl.ds(row_base + j * ZROWS, ZROWS)])

    def _write_out(block):
        out_off = block * NP + row_base
        for j in range(ROWS_PER_TILE // ZROWS):
            pltpu.sync_copy(acc.at[pl.ds(row_base + j * ZROWS, ZROWS)],
                            rows_a)
            pltpu.sync_copy(rows_a, agg_hbm.at[pl.ds(out_off + j * ZROWS,
                                                     ZROWS)])

    for p in range(R // NC):
        rep = NC * p + core
        idx_base = rep * N_EDGES + edge_base

        def _issue(i, sbuf, rbuf, sem):
            pltpu.sync_copy(srcoff_hbm.at[pl.ds(idx_base + i * CHUNK, CHUNK)],
                            sbuf)
            pltpu.async_copy(x_hbm.at[sbuf], rbuf, sem)

        def _waitg(sbuf, rbuf, sem):
            pltpu.make_async_copy(x_hbm.at[sbuf], rbuf, sem).wait()

        def _ascat(i, rbuf, sem):
            pltpu.async_copy(rbuf, acc.at[dstbig.at[i]], sem, add=True)

        def _waitsc(i, rbuf, sem):
            pltpu.make_async_copy(rbuf, acc.at[dstbig.at[i]], sem).wait()

        _zero_acc()
        plsc.subcore_barrier()

        _issue(0, src_a, rows_a, sem_a)
        _issue(1, src_b, rows_b, sem_b)

        def _chunk2(k, carry):
            _waitg(src_a, rows_a, sem_a)
            _ascat(2 * k, rows_a, sem_sa)
            _waitg(src_b, rows_b, sem_b)
            _ascat(2 * k + 1, rows_b, sem_sb)
            _waitsc(2 * k, rows_a, sem_sa)
            _issue(2 * k + 2, src_a, rows_a, sem_a)
            _waitsc(2 * k + 1, rows_b, sem_sb)
            _issue(2 * k + 3, src_b, rows_b, sem_b)
            return carry

        lax.fori_loop(0, (CHUNKS_PER_TILE - 3) // 2, _chunk2, 0)
        _waitg(src_a, rows_a, sem_a)
        _ascat(CHUNKS_PER_TILE - 3, rows_a, sem_sa)
        _waitg(src_b, rows_b, sem_b)
        _ascat(CHUNKS_PER_TILE - 2, rows_b, sem_sb)
        _waitsc(CHUNKS_PER_TILE - 3, rows_a, sem_sa)
        _issue(CHUNKS_PER_TILE - 1, src_a, rows_a, sem_a)
        _waitg(src_a, rows_a, sem_a)
        _ascat(CHUNKS_PER_TILE - 1, rows_a, sem_sa)
        _waitsc(CHUNKS_PER_TILE - 1, rows_a, sem_sa)
        _waitsc(CHUNKS_PER_TILE - 2, rows_b, sem_sb)
        plsc.subcore_barrier()
        _write_out(rep)

    _zero_acc()
    pltpu.sync_copy(ones_hbm, rows_a)
    plsc.subcore_barrier()

    def _dchunk(i, carry):
        pltpu.sync_copy(rows_a, acc.at[dstbig.at[i]], add=True)
        return carry

    lax.fori_loop(63 * core, 63 + core * (CHUNKS_PER_TILE - 63), _dchunk, 0)
    plsc.subcore_barrier()
    _write_out(R + core)


@functools.partial(
    pl.kernel,
    out_type=jax.ShapeDtypeStruct(((R + NC) * NP, F), jnp.float32),
    mesh=plsc.VectorSubcoreMesh(core_axis_name="c", subcore_axis_name="s"),
    scratch_types=[
        pltpu.VMEM_SHARED((NP, F), jnp.float32),
        pltpu.VMEM((CHUNK,), jnp.int32),
        pltpu.VMEM((CHUNK,), jnp.int32),
        pltpu.VMEM((CHUNKS_PER_TILE, CHUNK), jnp.int32),
        pltpu.VMEM((CHUNK, F), jnp.float32),
        pltpu.VMEM((CHUNK, F), jnp.float32),
        pltpu.SemaphoreType.DMA,
        pltpu.SemaphoreType.DMA,
        pltpu.SemaphoreType.DMA,
        pltpu.SemaphoreType.DMA,
    ],
)
def _sc_aggregate(*args):
    _sc_agg_kernel(*args)


BLK = 2000
T_DIM = 2


def _dense_body(x_ref, agg_ref, deg_ref, wn_ref, ws_ref, b_ref, o_ref):
    xb = x_ref[...][0]
    ab = agg_ref[...]
    deg = deg_ref[0, :, 0:1] + deg_ref[1, :, 0:1]
    inv = 1.0 / jnp.maximum(deg, 1.0)
    outs = []
    for t in range(T_DIM):
        acc = lax.dot_general(ab[t] * inv, wn_ref[...],
                              (((1,), (1,)), ((), ())),
                              preferred_element_type=jnp.float32)
        acc = acc + lax.dot_general(xb[:, t], ws_ref[...],
                                    (((1,), (1,)), ((), ())),
                                    preferred_element_type=jnp.float32)
        outs.append(jnp.maximum(acc + b_ref[...], 0.0))
    o_ref[...] = jnp.stack(outs, axis=1)[None]


def _dense(x4d, agg4d, deg3d, w_neigh, w_self, b2d):
    nblk = N_NODES // BLK
    nb = R // T_DIM
    return pl.pallas_call(
        _dense_body,
        grid=(nb, nblk),
        in_specs=[
            pl.BlockSpec((1, BLK, T_DIM, F), lambda b_, i: (b_, i, 0, 0)),
            pl.BlockSpec((T_DIM, BLK, F), lambda b_, i: (b_, i, 0)),
            pl.BlockSpec((NC, BLK, F), lambda b_, i: (R // NC, i, 0)),
            pl.BlockSpec((F, F), lambda b_, i: (0, 0)),
            pl.BlockSpec((F, F), lambda b_, i: (0, 0)),
            pl.BlockSpec((1, F), lambda b_, i: (0, 0)),
        ],
        out_specs=pl.BlockSpec((1, BLK, T_DIM, F),
                               lambda b_, i: (b_, i, 0, 0)),
        out_shape=jax.ShapeDtypeStruct(
            (R // T_DIM, N_NODES, T_DIM, F), jnp.float32),
    )(x4d, agg4d, deg3d, w_neigh, w_self, b2d)


def kernel(X, g, W_self, W_neigh, b):
    B, N, T, F_in = X.shape
    x_table = X.reshape(B * N * T, F_in)
    src = g[0]
    dst = g[1]
    rbase = ((jnp.arange(R, dtype=jnp.int32) // T) * (N * T)
             + jnp.arange(R, dtype=jnp.int32) % T)
    srcoff = (src[None, :] * T + rbase[:, None]).reshape(-1)
    zrow = jnp.zeros((ZROWS, F), jnp.float32)
    ones_c = jnp.ones((ZROWS, F), jnp.float32)
    out = _sc_aggregate(x_table, srcoff,
                        dst.reshape(NS, CHUNKS_PER_TILE, CHUNK), zrow,
                        ones_c)
    out6 = out.reshape(R + NC, NP, F)
    return _dense(X, out6, out6, W_neigh, W_self, b.reshape(1, F))


# speedup vs baseline: 1.0158x; 1.0158x over previous
import functools

import jax
import jax.numpy as jnp
from jax import lax
from jax.experimental import pallas as pl
from jax.experimental.pallas import tpu as pltpu
from jax.experimental.pallas import tpu_sc as plsc

N_NODES = 10000
NP = 10240
N_EDGES = 160000
F = 128
R = 4
NC = 2
NS = 16
CHUNK = 80
EDGES_PER_TILE = N_EDGES // NS
CHUNKS_PER_TILE = EDGES_PER_TILE // CHUNK
ROWS_PER_TILE = NP // NS
ZROWS = 80
DEG_CHUNKS = N_EDGES // CHUNK


def _sc_agg_kernel(x_hbm, srcoff_hbm, dst2_hbm, zrow_hbm, ones_hbm,
                   agg_hbm,
                   acc, src_a, src_b, dstbig, rows_a, rows_b,
                   sem_a, sem_b):
    core = lax.axis_index("c")
    sub = lax.axis_index("s")
    edge_base = sub * EDGES_PER_TILE
    row_base = sub * ROWS_PER_TILE

    pltpu.sync_copy(dst2_hbm.at[sub], dstbig)

    def _zero_acc():
        pltpu.sync_copy(zrow_hbm, rows_a)
        for j in range(ROWS_PER_TILE // ZROWS):
            pltpu.sync_copy(rows_a,
                            acc.at[pl.ds(row_base + j * ZROWS, ZROWS)])

    def _write_out(block):
        out_off = block * NP + row_base
        for j in range(ROWS_PER_TILE // ZROWS):
            pltpu.sync_copy(acc.at[pl.ds(row_base + j * ZROWS, ZROWS)],
                            rows_a)
            pltpu.sync_copy(rows_a, agg_hbm.at[pl.ds(out_off + j * ZROWS,
                                                     ZROWS)])

    for p in range(R // NC):
        rep = NC * p + core
        idx_base = rep * N_EDGES + edge_base

        def _issue(i, sbuf, rbuf, sem):
            pltpu.sync_copy(srcoff_hbm.at[pl.ds(idx_base + i * CHUNK, CHUNK)],
                            sbuf)
            pltpu.async_copy(x_hbm.at[sbuf], rbuf, sem)

        def _drain(i, sbuf, rbuf, sem):
            pltpu.make_async_copy(x_hbm.at[sbuf], rbuf, sem).wait()
            pltpu.sync_copy(rbuf, acc.at[dstbig.at[i]], add=True)

        _zero_acc()
        plsc.subcore_barrier()

        _issue(0, src_a, rows_a, sem_a)
        _issue(1, src_b, rows_b, sem_b)

        def _chunk2(k, carry):
            _drain(2 * k, src_a, rows_a, sem_a)
            _issue(2 * k + 2, src_a, rows_a, sem_a)
            _drain(2 * k + 1, src_b, rows_b, sem_b)
            _issue(2 * k + 3, src_b, rows_b, sem_b)
            return carry

        lax.fori_loop(0, (CHUNKS_PER_TILE - 3) // 2, _chunk2, 0)
        _drain(CHUNKS_PER_TILE - 3, src_a, rows_a, sem_a)
        _drain(CHUNKS_PER_TILE - 2, src_b, rows_b, sem_b)
        _issue(CHUNKS_PER_TILE - 1, src_a, rows_a, sem_a)
        _drain(CHUNKS_PER_TILE - 1, src_a, rows_a, sem_a)
        plsc.subcore_barrier()
        _write_out(rep)

    _zero_acc()
    pltpu.sync_copy(ones_hbm, rows_a)
    plsc.subcore_barrier()

    def _dchunk(i, carry):
        pltpu.sync_copy(rows_a, acc.at[dstbig.at[i]], add=True)
        return carry

    lax.fori_loop(63 * core, 63 + core * (CHUNKS_PER_TILE - 63), _dchunk, 0)
    plsc.subcore_barrier()
    _write_out(R + core)


@functools.partial(
    pl.kernel,
    out_type=jax.ShapeDtypeStruct(((R + NC) * NP, F), jnp.float32),
    mesh=plsc.VectorSubcoreMesh(core_axis_name="c", subcore_axis_name="s"),
    scratch_types=[
        pltpu.VMEM_SHARED((NP, F), jnp.float32),
        pltpu.VMEM((CHUNK,), jnp.int32),
        pltpu.VMEM((CHUNK,), jnp.int32),
        pltpu.VMEM((CHUNKS_PER_TILE, CHUNK), jnp.int32),
        pltpu.VMEM((CHUNK, F), jnp.float32),
        pltpu.VMEM((CHUNK, F), jnp.float32),
        pltpu.SemaphoreType.DMA,
        pltpu.SemaphoreType.DMA,
    ],
)
def _sc_aggregate(*args):
    _sc_agg_kernel(*args)


BLK = 2000
T_DIM = 2


def _dense_body(x_ref, agg_ref, deg_ref, wn_ref, ws_ref, b_ref, o_ref):
    xb = x_ref[...][0]
    ab = agg_ref[...]
    deg = deg_ref[0, :, 0:1] + deg_ref[1, :, 0:1]
    inv = 1.0 / jnp.maximum(deg, 1.0)
    outs = []
    for t in range(T_DIM):
        acc = lax.dot_general(ab[t] * inv, wn_ref[...],
                              (((1,), (1,)), ((), ())),
                              preferred_element_type=jnp.float32)
        acc = acc + lax.dot_general(xb[:, t], ws_ref[...],
                                    (((1,), (1,)), ((), ())),
                                    preferred_element_type=jnp.float32)
        outs.append(jnp.maximum(acc + b_ref[...], 0.0))
    o_ref[...] = jnp.stack(outs, axis=1)[None]


def _dense(x4d, agg4d, deg3d, w_neigh, w_self, b2d):
    nblk = N_NODES // BLK
    nb = R // T_DIM
    return pl.pallas_call(
        _dense_body,
        grid=(nb, nblk),
        in_specs=[
            pl.BlockSpec((1, BLK, T_DIM, F), lambda b_, i: (b_, i, 0, 0)),
            pl.BlockSpec((T_DIM, BLK, F), lambda b_, i: (b_, i, 0)),
            pl.BlockSpec((NC, BLK, F), lambda b_, i: (R // NC, i, 0)),
            pl.BlockSpec((F, F), lambda b_, i: (0, 0)),
            pl.BlockSpec((F, F), lambda b_, i: (0, 0)),
            pl.BlockSpec((1, F), lambda b_, i: (0, 0)),
        ],
        out_specs=pl.BlockSpec((1, BLK, T_DIM, F),
                               lambda b_, i: (b_, i, 0, 0)),
        out_shape=jax.ShapeDtypeStruct(
            (R // T_DIM, N_NODES, T_DIM, F), jnp.float32),
    )(x4d, agg4d, deg3d, w_neigh, w_self, b2d)


def kernel(X, g, W_self, W_neigh, b):
    B, N, T, F_in = X.shape
    x_table = X.reshape(B * N * T, F_in)
    src = g[0]
    dst = g[1]
    rbase = ((jnp.arange(R, dtype=jnp.int32) // T) * (N * T)
             + jnp.arange(R, dtype=jnp.int32) % T)
    srcoff = (src[None, :] * T + rbase[:, None]).reshape(-1)
    zrow = jnp.zeros((ZROWS, F), jnp.float32)
    ones_c = jnp.ones((ZROWS, F), jnp.float32)
    out = _sc_aggregate(x_table, srcoff,
                        dst.reshape(NS, CHUNKS_PER_TILE, CHUNK), zrow,
                        ones_c)
    out6 = out.reshape(R + NC, NP, F)
    return _dense(X, out6, out6, W_neigh, W_self, b.reshape(1, F))
